# Initial kernel scaffold; baseline (speedup 1.0000x reference)
import functools

import jax
import jax.numpy as jnp
from jax import lax
from jax.experimental import pallas as pl
from jax.experimental.pallas import tpu as pltpu
from jax.experimental.pallas import tpu_sc as plsc

N = 100000
NP = 102400
F = 20
H = 16
CLS = 2
NC = 2
NS = 16
NW = NC * NS
CHUNK = 128
NPT = NP // NS
NPW = NP // NW
BN = 1024


def _mesh():
    return plsc.VectorSubcoreMesh(core_axis_name="c", subcore_axis_name="s",
                                  num_cores=NC, num_subcores=NS)


def _worker():
    c = lax.axis_index("c")
    s = lax.axis_index("s")
    return c, s


def _fill(ref, n, value, dtype):
    v = jnp.full((16,), value, dtype)

    @pl.loop(0, n // 16)
    def _(k):
        ref[pl.ds(k * 16, 16)] = v


def _k1a_body(nch, col_hbm, part_hbm, cbuf, ones_b, zbuf, dacc, sem):
    c, s = _worker()
    _fill(ones_b, CHUNK, 1.0, jnp.float32)
    _fill(zbuf, CHUNK, 0.0, jnp.float32)

    @pl.loop(0, NPT // CHUNK)
    def _(j):
        pltpu.sync_copy(zbuf, dacc.at[pl.ds(s * NPT + j * CHUNK, CHUNK)])

    plsc.subcore_barrier()
    base = (c * NS + s) * (nch * CHUNK)

    @pl.loop(0, nch)
    def _(i):
        pltpu.sync_copy(col_hbm.at[pl.ds(base + i * CHUNK, CHUNK)], cbuf)
        pltpu.async_copy(ones_b, dacc.at[cbuf], sem, add=True).wait()

    plsc.subcore_barrier()

    @pl.loop(0, NPT // CHUNK)
    def _(j):
        off = s * NPT + j * CHUNK
        pltpu.sync_copy(dacc.at[pl.ds(off, CHUNK)], zbuf)
        pltpu.sync_copy(zbuf, part_hbm.at[c, pl.ds(off, CHUNK)])


def _k1b_body(part_hbm, dinv16_hbm, p0b, p1b, ytmp, ob):
    c, s = _worker()
    w = c * NS + s
    pltpu.sync_copy(part_hbm.at[0, pl.ds(w * NPW, NPW)], p0b)
    pltpu.sync_copy(part_hbm.at[1, pl.ds(w * NPW, NPW)], p1b)

    @pl.loop(0, NPW // 16)
    def _(g):
        d = p0b[pl.ds(g * 16, 16)] + p1b[pl.ds(g * 16, 16)] + 1.0
        i = plsc.bitcast(d, jnp.int32)
        y = plsc.bitcast(jnp.int32(0x5F3759DF) - (i >> 1), jnp.float32)
        for _ in range(3):
            y = y * (1.5 - 0.5 * d * y * y)
        ytmp[...] = y
        for j in range(16):
            ob[g * 16 + j] = plsc.load_gather(
                ytmp, [jnp.full((16,), j, jnp.int32)])

    pltpu.sync_copy(ob, dinv16_hbm.at[pl.ds(w * NPW, NPW)])


def _k3_body(nch, row_hbm, col_hbm, h1p_hbm, out_hbm,
             rbuf, cbuf, gbuf, zbuf16, acc, semg, sems):
    c, s = _worker()

    @pl.loop(0, CHUNK)
    def _(k):
        zbuf16[k] = jnp.zeros((16,), jnp.float32)

    @pl.loop(0, NPT // CHUNK)
    def _(j):
        pltpu.sync_copy(zbuf16, acc.at[pl.ds(s * NPT + j * CHUNK, CHUNK)])

    plsc.subcore_barrier()
    base = (c * NS + s) * (nch * CHUNK)

    @pl.loop(0, nch)
    def _(i):
        pltpu.sync_copy(row_hbm.at[pl.ds(base + i * CHUNK, CHUNK)], rbuf)
        pltpu.sync_copy(col_hbm.at[pl.ds(base + i * CHUNK, CHUNK)], cbuf)
        pltpu.async_copy(h1p_hbm.at[rbuf], gbuf, semg).wait()
        pltpu.async_copy(gbuf, acc.at[cbuf], sems, add=True).wait()

    plsc.subcore_barrier()

    @pl.loop(0, NPT // CHUNK)
    def _(j):
        off = s * NPT + j * CHUNK
        pltpu.sync_copy(acc.at[pl.ds(off, CHUNK)], zbuf16)
        pltpu.sync_copy(zbuf16, out_hbm.at[c, pl.ds(off, CHUNK)])


def _k5_body(nch, row_hbm, col_hbm, h2f_hbm, out_hbm,
             rbuf, cbuf, r2a, r2b, c2a, c2b, ga, gb, zbuf, acc2, semg, sems):
    c, s = _worker()
    _fill(zbuf, CHUNK, 0.0, jnp.float32)
    npt2 = 2 * NP // NS

    @pl.loop(0, npt2 // CHUNK)
    def _(j):
        pltpu.sync_copy(zbuf, acc2.at[pl.ds(s * npt2 + j * CHUNK, CHUNK)])

    plsc.subcore_barrier()
    base = (c * NS + s) * (nch * CHUNK)

    @pl.loop(0, nch)
    def _(i):
        pltpu.sync_copy(row_hbm.at[pl.ds(base + i * CHUNK, CHUNK)], rbuf)
        pltpu.sync_copy(col_hbm.at[pl.ds(base + i * CHUNK, CHUNK)], cbuf)

        @pl.loop(0, CHUNK // 16)
        def _(k):
            sl = pl.ds(k * 16, 16)
            rv = rbuf[sl] * 2
            r2a[sl] = rv
            r2b[sl] = rv + 1
            cv = cbuf[sl] * 2
            c2a[sl] = cv
            c2b[sl] = cv + 1

        pltpu.async_copy(h2f_hbm.at[r2a], ga, semg).wait()
        pltpu.async_copy(h2f_hbm.at[r2b], gb, semg).wait()
        pltpu.async_copy(ga, acc2.at[c2a], sems, add=True).wait()
        pltpu.async_copy(gb, acc2.at[c2b], sems, add=True).wait()

    plsc.subcore_barrier()

    @pl.loop(0, npt2 // CHUNK)
    def _(j):
        off = s * npt2 + j * CHUNK
        pltpu.sync_copy(acc2.at[pl.ds(off, CHUNK)], zbuf)
        pltpu.sync_copy(zbuf, out_hbm.at[c, pl.ds(off, CHUNK)])


def _k2_body(x_ref, w1_ref, dinv_ref, o_ref):
    t1 = jnp.dot(x_ref[...], w1_ref[...], preferred_element_type=jnp.float32)
    o_ref[...] = dinv_ref[...] * t1


def _k4_body(acc_ref, h1p_ref, dinv_ref, b1_ref, w2_ref, o_ref):
    dinv = dinv_ref[...]
    agg = acc_ref[0] + acc_ref[1] + h1p_ref[...]
    z = jnp.maximum(dinv * agg + b1_ref[...], 0.0)
    t2 = jnp.dot(z, w2_ref[...], preferred_element_type=jnp.float32)
    o_ref[...] = dinv[:, :CLS] * t2


def _k6_body(acc_ref, h2p_ref, dinv_ref, b2_ref, o_ref):
    o = dinv_ref[...][:, :CLS] * (acc_ref[0] + acc_ref[1] + h2p_ref[...])
    o = o + b2_ref[...]
    m = jnp.max(o, axis=1, keepdims=True)
    ssum = jnp.sum(jnp.exp(o - m), axis=1, keepdims=True)
    o_ref[...] = o - m - jnp.log(ssum)


@jax.jit
def kernel(x, edge_index, W1, b1, W2, b2):
    f32 = jnp.float32
    row = edge_index[0].astype(jnp.int32)
    col = edge_index[1].astype(jnp.int32)
    e = row.shape[0]
    ep = ((e + NW * CHUNK - 1) // (NW * CHUNK)) * (NW * CHUNK)
    nch = ep // (NW * CHUNK)
    npad = ep - e
    padi = N + (jnp.arange(npad, dtype=jnp.int32) % 128)
    row_p = jnp.concatenate([row, padi])
    col_p = jnp.concatenate([col, padi])
    x_p = jnp.pad(x, ((0, NP - N), (0, 0)))

    k1a = pl.kernel(
        functools.partial(_k1a_body, nch),
        out_type=jax.ShapeDtypeStruct((NC, NP), f32),
        mesh=_mesh(),
        scratch_types=[
            pltpu.VMEM((CHUNK,), jnp.int32),
            pltpu.VMEM((CHUNK,), f32),
            pltpu.VMEM((CHUNK,), f32),
            pltpu.VMEM_SHARED((NP,), f32),
            pltpu.SemaphoreType.DMA,
        ],
    )
    deg_part = k1a(col_p)

    k1b = pl.kernel(
        _k1b_body,
        out_type=jax.ShapeDtypeStruct((NP, H), f32),
        mesh=_mesh(),
        scratch_types=[
            pltpu.VMEM((NPW,), f32),
            pltpu.VMEM((NPW,), f32),
            pltpu.VMEM((16,), f32),
            pltpu.VMEM((NPW, H), f32),
        ],
    )
    dinv16 = k1b(deg_part)

    h1p = pl.pallas_call(
        _k2_body, grid=(NP // BN,),
        in_specs=[pl.BlockSpec((BN, F), lambda i: (i, 0)),
                  pl.BlockSpec((F, H), lambda i: (0, 0)),
                  pl.BlockSpec((BN, H), lambda i: (i, 0))],
        out_specs=pl.BlockSpec((BN, H), lambda i: (i, 0)),
        out_shape=jax.ShapeDtypeStruct((NP, H), f32),
    )(x_p, W1, dinv16)

    k3 = pl.kernel(
        functools.partial(_k3_body, nch),
        out_type=jax.ShapeDtypeStruct((NC, NP, H), f32),
        mesh=_mesh(),
        scratch_types=[
            pltpu.VMEM((CHUNK,), jnp.int32),
            pltpu.VMEM((CHUNK,), jnp.int32),
            pltpu.VMEM((CHUNK, H), f32),
            pltpu.VMEM((CHUNK, H), f32),
            pltpu.VMEM_SHARED((NP, H), f32),
            pltpu.SemaphoreType.DMA,
            pltpu.SemaphoreType.DMA,
        ],
    )
    acc1 = k3(row_p, col_p, h1p)

    h2p = pl.pallas_call(
        _k4_body, grid=(NP // BN,),
        in_specs=[pl.BlockSpec((NC, BN, H), lambda i: (0, i, 0)),
                  pl.BlockSpec((BN, H), lambda i: (i, 0)),
                  pl.BlockSpec((BN, H), lambda i: (i, 0)),
                  pl.BlockSpec((1, H), lambda i: (0, 0)),
                  pl.BlockSpec((H, CLS), lambda i: (0, 0))],
        out_specs=pl.BlockSpec((BN, CLS), lambda i: (i, 0)),
        out_shape=jax.ShapeDtypeStruct((NP, CLS), f32),
    )(acc1, h1p, dinv16, b1.reshape(1, H), W2)

    h2f = h2p.reshape(-1)
    k5 = pl.kernel(
        functools.partial(_k5_body, nch),
        out_type=jax.ShapeDtypeStruct((NC, 2 * NP), f32),
        mesh=_mesh(),
        scratch_types=[
            pltpu.VMEM((CHUNK,), jnp.int32),
            pltpu.VMEM((CHUNK,), jnp.int32),
            pltpu.VMEM((CHUNK,), jnp.int32),
            pltpu.VMEM((CHUNK,), jnp.int32),
            pltpu.VMEM((CHUNK,), jnp.int32),
            pltpu.VMEM((CHUNK,), jnp.int32),
            pltpu.VMEM((CHUNK,), f32),
            pltpu.VMEM((CHUNK,), f32),
            pltpu.VMEM((CHUNK,), f32),
            pltpu.VMEM_SHARED((2 * NP,), f32),
            pltpu.SemaphoreType.DMA,
            pltpu.SemaphoreType.DMA,
        ],
    )
    acc2 = k5(row_p, col_p, h2f).reshape(NC, NP, CLS)

    out = pl.pallas_call(
        _k6_body, grid=(NP // BN,),
        in_specs=[pl.BlockSpec((NC, BN, CLS), lambda i: (0, i, 0)),
                  pl.BlockSpec((BN, CLS), lambda i: (i, 0)),
                  pl.BlockSpec((BN, H), lambda i: (i, 0)),
                  pl.BlockSpec((1, CLS), lambda i: (0, 0))],
        out_specs=pl.BlockSpec((BN, CLS), lambda i: (i, 0)),
        out_shape=jax.ShapeDtypeStruct((NP, CLS), f32),
    )(acc2, h2p, dinv16, b2.reshape(1, CLS))

    return out[:N]


# speedup vs baseline: 17.6091x; 17.6091x over previous
import functools

import jax
import jax.numpy as jnp
from jax import lax
from jax.experimental import pallas as pl
from jax.experimental.pallas import tpu as pltpu
from jax.experimental.pallas import tpu_sc as plsc

N = 100000
NP = 102400
F = 20
H = 16
CLS = 2
NC = 2
NS = 16
NW = NC * NS
CHUNK = 128
NPT = NP // NS
NPW = NP // NW
BN = 1024


def _mesh():
    return plsc.VectorSubcoreMesh(core_axis_name="c", subcore_axis_name="s",
                                  num_cores=NC, num_subcores=NS)


def _worker():
    c = lax.axis_index("c")
    s = lax.axis_index("s")
    return c, s


def _fill(ref, n, value, dtype):
    v = jnp.full((16,), value, dtype)

    @pl.loop(0, n // 16)
    def _(k):
        ref[pl.ds(k * 16, 16)] = v


def _k1a_body(nch, col_hbm, part_hbm, cbuf, ones_b, zbuf, dacc, sem):
    c, s = _worker()
    _fill(ones_b, CHUNK, 1.0, jnp.float32)
    _fill(zbuf, CHUNK, 0.0, jnp.float32)

    @pl.loop(0, NPT // CHUNK)
    def _(j):
        pltpu.sync_copy(zbuf, dacc.at[pl.ds(s * NPT + j * CHUNK, CHUNK)])

    plsc.subcore_barrier()
    base = (c * NS + s) * (nch * CHUNK)

    @pl.loop(0, nch)
    def _(i):
        pltpu.sync_copy(col_hbm.at[pl.ds(base + i * CHUNK, CHUNK)], cbuf)
        pltpu.async_copy(ones_b, dacc.at[cbuf], sem, add=True).wait()

    plsc.subcore_barrier()

    @pl.loop(0, NPT // CHUNK)
    def _(j):
        off = s * NPT + j * CHUNK
        pltpu.sync_copy(dacc.at[pl.ds(off, CHUNK)], zbuf)
        pltpu.sync_copy(zbuf, part_hbm.at[c, pl.ds(off, CHUNK)])


def _k1b_body(part_hbm, dinv16_hbm, p0b, p1b, ytmp, ob):
    c, s = _worker()
    w = c * NS + s
    pltpu.sync_copy(part_hbm.at[0, pl.ds(w * NPW, NPW)], p0b)
    pltpu.sync_copy(part_hbm.at[1, pl.ds(w * NPW, NPW)], p1b)

    @pl.loop(0, NPW // 16)
    def _(g):
        d = p0b[pl.ds(g * 16, 16)] + p1b[pl.ds(g * 16, 16)] + 1.0
        i = plsc.bitcast(d, jnp.int32)
        y = plsc.bitcast(jnp.int32(0x5F3759DF) - (i >> 1), jnp.float32)
        for _ in range(3):
            y = y * (1.5 - 0.5 * d * y * y)
        ytmp[...] = y
        for j in range(16):
            ob[g * 16 + j] = plsc.load_gather(
                ytmp, [jnp.full((16,), j, jnp.int32)])

    pltpu.sync_copy(ob, dinv16_hbm.at[pl.ds(w * NPW, NPW)])


def _k3_body(nch, row_hbm, col_hbm, h1p_hbm, out_hbm,
             rbuf, cbuf, gbuf, zbuf16, acc, semg, sems):
    c, s = _worker()

    @pl.loop(0, CHUNK)
    def _(k):
        zbuf16[k] = jnp.zeros((16,), jnp.float32)

    @pl.loop(0, NPT // CHUNK)
    def _(j):
        pltpu.sync_copy(zbuf16, acc.at[pl.ds(s * NPT + j * CHUNK, CHUNK)])

    plsc.subcore_barrier()
    base = (c * NS + s) * (nch * CHUNK)

    @pl.loop(0, nch)
    def _(i):
        pltpu.sync_copy(row_hbm.at[pl.ds(base + i * CHUNK, CHUNK)], rbuf)
        pltpu.sync_copy(col_hbm.at[pl.ds(base + i * CHUNK, CHUNK)], cbuf)
        pltpu.async_copy(h1p_hbm.at[rbuf], gbuf, semg).wait()
        pltpu.async_copy(gbuf, acc.at[cbuf], sems, add=True).wait()

    plsc.subcore_barrier()

    @pl.loop(0, NPT // CHUNK)
    def _(j):
        off = s * NPT + j * CHUNK
        pltpu.sync_copy(acc.at[pl.ds(off, CHUNK)], zbuf16)
        pltpu.sync_copy(zbuf16, out_hbm.at[c, pl.ds(off, CHUNK)])


def _k5_body(nch, row_hbm, col_hbm, h2f_hbm, out_hbm,
             rbuf, cbuf, r2a, r2b, c2a, c2b, ga, gb, zbuf, acc2, semg, sems):
    c, s = _worker()
    _fill(zbuf, CHUNK, 0.0, jnp.float32)
    npt2 = 2 * NP // NS

    @pl.loop(0, npt2 // CHUNK)
    def _(j):
        pltpu.sync_copy(zbuf, acc2.at[pl.ds(s * npt2 + j * CHUNK, CHUNK)])

    plsc.subcore_barrier()
    base = (c * NS + s) * (nch * CHUNK)

    @pl.loop(0, nch)
    def _(i):
        pltpu.sync_copy(row_hbm.at[pl.ds(base + i * CHUNK, CHUNK)], rbuf)
        pltpu.sync_copy(col_hbm.at[pl.ds(base + i * CHUNK, CHUNK)], cbuf)

        @pl.loop(0, CHUNK // 16)
        def _(k):
            sl = pl.ds(k * 16, 16)
            rv = rbuf[sl] * 2
            r2a[sl] = rv
            r2b[sl] = rv + 1
            cv = cbuf[sl] * 2
            c2a[sl] = cv
            c2b[sl] = cv + 1

        pltpu.async_copy(h2f_hbm.at[r2a], ga, semg).wait()
        pltpu.async_copy(h2f_hbm.at[r2b], gb, semg).wait()
        pltpu.async_copy(ga, acc2.at[c2a], sems, add=True).wait()
        pltpu.async_copy(gb, acc2.at[c2b], sems, add=True).wait()

    plsc.subcore_barrier()

    @pl.loop(0, npt2 // CHUNK)
    def _(j):
        off = s * npt2 + j * CHUNK
        pltpu.sync_copy(acc2.at[pl.ds(off, CHUNK)], zbuf)
        pltpu.sync_copy(zbuf, out_hbm.at[c, pl.ds(off, CHUNK)])


def _k2_body(x_ref, w1_ref, dinv_ref, o_ref):
    t1 = jnp.dot(x_ref[...], w1_ref[...], preferred_element_type=jnp.float32)
    o_ref[...] = dinv_ref[...] * t1


def _k4_body(acc_ref, h1p_ref, dinv_ref, b1_ref, w2_ref, o_ref):
    dinv = dinv_ref[...]
    agg = acc_ref[0] + acc_ref[1] + h1p_ref[...]
    z = jnp.maximum(dinv * agg + b1_ref[...], 0.0)
    t2 = jnp.dot(z, w2_ref[...], preferred_element_type=jnp.float32)
    o_ref[...] = dinv[:, :CLS] * t2


def _k6_body(acc_ref, h2p_ref, dinv_ref, b2_ref, o_ref):
    o = dinv_ref[...][:, :CLS] * (acc_ref[0] + acc_ref[1] + h2p_ref[...])
    o = o + b2_ref[...]
    m = jnp.max(o, axis=1, keepdims=True)
    ssum = jnp.sum(jnp.exp(o - m), axis=1, keepdims=True)
    o_ref[...] = o - m - jnp.log(ssum)


@jax.jit
def kernel(x, edge_index, W1, b1, W2, b2):
    f32 = jnp.float32
    row = edge_index[0].astype(jnp.int32)
    col = edge_index[1].astype(jnp.int32)
    e = row.shape[0]
    ep = ((e + NW * CHUNK - 1) // (NW * CHUNK)) * (NW * CHUNK)
    nch = ep // (NW * CHUNK)
    npad = ep - e
    padi = N + (jnp.arange(npad, dtype=jnp.int32) % 128)
    row_p = jnp.concatenate([row, padi])
    col_p = jnp.concatenate([col, padi])
    x_p = jnp.pad(x, ((0, NP - N), (0, 0)))

    k1a = pl.kernel(
        functools.partial(_k1a_body, nch),
        out_type=jax.ShapeDtypeStruct((NC, NP), f32),
        mesh=_mesh(),
        compiler_params=pltpu.CompilerParams(needs_layout_passes=False, use_tc_tiling_on_sc=False),
        scratch_types=[
            pltpu.VMEM((CHUNK,), jnp.int32),
            pltpu.VMEM((CHUNK,), f32),
            pltpu.VMEM((CHUNK,), f32),
            pltpu.VMEM_SHARED((NP,), f32),
            pltpu.SemaphoreType.DMA,
        ],
    )
    deg_part = k1a(col_p)

    k1b = pl.kernel(
        _k1b_body,
        out_type=jax.ShapeDtypeStruct((NP, H), f32),
        mesh=_mesh(),
        compiler_params=pltpu.CompilerParams(needs_layout_passes=False, use_tc_tiling_on_sc=False),
        scratch_types=[
            pltpu.VMEM((NPW,), f32),
            pltpu.VMEM((NPW,), f32),
            pltpu.VMEM((16,), f32),
            pltpu.VMEM((NPW, H), f32),
        ],
    )
    dinv16 = k1b(deg_part)

    h1p = pl.pallas_call(
        _k2_body, grid=(NP // BN,),
        in_specs=[pl.BlockSpec((BN, F), lambda i: (i, 0)),
                  pl.BlockSpec((F, H), lambda i: (0, 0)),
                  pl.BlockSpec((BN, H), lambda i: (i, 0))],
        out_specs=pl.BlockSpec((BN, H), lambda i: (i, 0)),
        out_shape=jax.ShapeDtypeStruct((NP, H), f32),
    )(x_p, W1, dinv16)

    k3 = pl.kernel(
        functools.partial(_k3_body, nch),
        out_type=jax.ShapeDtypeStruct((NC, NP, H), f32),
        mesh=_mesh(),
        compiler_params=pltpu.CompilerParams(needs_layout_passes=False, use_tc_tiling_on_sc=False),
        scratch_types=[
            pltpu.VMEM((CHUNK,), jnp.int32),
            pltpu.VMEM((CHUNK,), jnp.int32),
            pltpu.VMEM((CHUNK, H), f32),
            pltpu.VMEM((CHUNK, H), f32),
            pltpu.VMEM_SHARED((NP, H), f32),
            pltpu.SemaphoreType.DMA,
            pltpu.SemaphoreType.DMA,
        ],
    )
    acc1 = k3(row_p, col_p, h1p)

    h2p = pl.pallas_call(
        _k4_body, grid=(NP // BN,),
        in_specs=[pl.BlockSpec((NC, BN, H), lambda i: (0, i, 0)),
                  pl.BlockSpec((BN, H), lambda i: (i, 0)),
                  pl.BlockSpec((BN, H), lambda i: (i, 0)),
                  pl.BlockSpec((1, H), lambda i: (0, 0)),
                  pl.BlockSpec((H, CLS), lambda i: (0, 0))],
        out_specs=pl.BlockSpec((BN, CLS), lambda i: (i, 0)),
        out_shape=jax.ShapeDtypeStruct((NP, CLS), f32),
    )(acc1, h1p, dinv16, b1.reshape(1, H), W2)

    h2f = h2p.reshape(-1)
    k5 = pl.kernel(
        functools.partial(_k5_body, nch),
        out_type=jax.ShapeDtypeStruct((NC, 2 * NP), f32),
        mesh=_mesh(),
        compiler_params=pltpu.CompilerParams(needs_layout_passes=False, use_tc_tiling_on_sc=False),
        scratch_types=[
            pltpu.VMEM((CHUNK,), jnp.int32),
            pltpu.VMEM((CHUNK,), jnp.int32),
            pltpu.VMEM((CHUNK,), jnp.int32),
            pltpu.VMEM((CHUNK,), jnp.int32),
            pltpu.VMEM((CHUNK,), jnp.int32),
            pltpu.VMEM((CHUNK,), jnp.int32),
            pltpu.VMEM((CHUNK,), f32),
            pltpu.VMEM((CHUNK,), f32),
            pltpu.VMEM((CHUNK,), f32),
            pltpu.VMEM_SHARED((2 * NP,), f32),
            pltpu.SemaphoreType.DMA,
            pltpu.SemaphoreType.DMA,
        ],
    )
    acc2 = k5(row_p, col_p, h2f).reshape(NC, NP, CLS)

    out = pl.pallas_call(
        _k6_body, grid=(NP // BN,),
        in_specs=[pl.BlockSpec((NC, BN, CLS), lambda i: (0, i, 0)),
                  pl.BlockSpec((BN, CLS), lambda i: (i, 0)),
                  pl.BlockSpec((BN, H), lambda i: (i, 0)),
                  pl.BlockSpec((1, CLS), lambda i: (0, 0))],
        out_specs=pl.BlockSpec((BN, CLS), lambda i: (i, 0)),
        out_shape=jax.ShapeDtypeStruct((NP, CLS), f32),
    )(acc2, h2p, dinv16, b2.reshape(1, CLS))

    return out[:N]


# speedup vs baseline: 45.9248x; 2.6080x over previous
import functools

import jax
import jax.numpy as jnp
from jax import lax
from jax.experimental import pallas as pl
from jax.experimental.pallas import tpu as pltpu
from jax.experimental.pallas import tpu_sc as plsc

N = 100000
NP = 102400
F = 20
H = 16
CLS = 2
NC = 2
NS = 16
NW = NC * NS
CHUNK = 128
G = 4
NPT = NP // NS
NPW = NP // NW
BN = 1024

_SC_PARAMS = pltpu.CompilerParams(needs_layout_passes=False,
                                  use_tc_tiling_on_sc=False)


def _mesh():
    return plsc.VectorSubcoreMesh(core_axis_name="c", subcore_axis_name="s",
                                  num_cores=NC, num_subcores=NS)


def _worker():
    c = lax.axis_index("c")
    s = lax.axis_index("s")
    return c, s


def _fill(ref, n, value, dtype):
    v = jnp.full((16,), value, dtype)

    @pl.loop(0, n // 16)
    def _(k):
        ref[pl.ds(k * 16, 16)] = v


def _k1a_body(nblk, eidx_hbm, z1_hbm, part_hbm,
              ib0, ib1, ones_b, bb, dacc, sem0, sem1):
    c, s = _worker()
    w = c * NS + s
    _fill(ones_b, CHUNK, 1.0, jnp.float32)
    _fill(bb, NPT // 8, 0.0, jnp.float32)

    @pl.loop(0, 8)
    def _(t):
        pltpu.sync_copy(bb, dacc.at[pl.ds(s * NPT + t * (NPT // 8),
                                          NPT // 8)])

    plsc.subcore_barrier()

    def block(j, ib, sem):
        @pl.when(j >= 2)
        def _():
            for _g in range(G):
                pltpu.make_async_copy(
                    z1_hbm.at[pl.ds(0, CHUNK)], ones_b, sem).wait()

        pltpu.sync_copy(eidx_hbm.at[w, j], ib)
        for g in range(G):
            pltpu.async_copy(ones_b, dacc.at[ib.at[G + g]], sem, add=True)

    @pl.loop(0, nblk)
    def _(j):
        @pl.when(j % 2 == 0)
        def _():
            block(j, ib0, sem0)

        @pl.when(j % 2 == 1)
        def _():
            block(j, ib1, sem1)

    for sem in (sem0, sem1):
        for _g in range(G):
            pltpu.make_async_copy(
                z1_hbm.at[pl.ds(0, CHUNK)], ones_b, sem).wait()

    plsc.subcore_barrier()

    @pl.loop(0, 8)
    def _(t):
        off = s * NPT + t * (NPT // 8)
        pltpu.sync_copy(dacc.at[pl.ds(off, NPT // 8)], bb)
        pltpu.sync_copy(bb, part_hbm.at[c, pl.ds(off, NPT // 8)])


def _k1b_body(part_hbm, dinv16_hbm, p0b, p1b, ytmp, ob):
    c, s = _worker()
    w = c * NS + s
    pltpu.sync_copy(part_hbm.at[0, pl.ds(w * NPW, NPW)], p0b)
    pltpu.sync_copy(part_hbm.at[1, pl.ds(w * NPW, NPW)], p1b)

    @pl.loop(0, NPW // 16)
    def _(g):
        d = p0b[pl.ds(g * 16, 16)] + p1b[pl.ds(g * 16, 16)] + 1.0
        i = plsc.bitcast(d, jnp.int32)
        y = plsc.bitcast(jnp.int32(0x5F3759DF) - (i >> 1), jnp.float32)
        for _ in range(3):
            y = y * (1.5 - 0.5 * d * y * y)
        ytmp[...] = y
        for j in range(16):
            ob[g * 16 + j] = plsc.load_gather(
                ytmp, [jnp.full((16,), j, jnp.int32)])

    pltpu.sync_copy(ob, dinv16_hbm.at[pl.ds(w * NPW, NPW)])


def _make_prop_body(nblk, d):
    def body(eidx_hbm, zd_hbm, h_hbm, out_hbm,
             ib0, ib1, gb0, gb1, acc, semg0, semg1, sems0, sems1):
        c, s = _worker()
        w = c * NS + s
        pltpu.sync_copy(zd_hbm.at[pl.ds(0, CHUNK)], gb0.at[pl.ds(0, CHUNK)])

        @pl.loop(0, NPT // CHUNK)
        def _(j):
            pltpu.sync_copy(gb0.at[pl.ds(0, CHUNK)],
                            acc.at[pl.ds(s * NPT + j * CHUNK, CHUNK)])

        plsc.subcore_barrier()

        def block(j, ib, gb, semg, sems):
            @pl.when(j >= 2)
            def _():
                for g in range(G):
                    pltpu.make_async_copy(
                        h_hbm.at[pl.ds(0, CHUNK)],
                        gb.at[pl.ds(g * CHUNK, CHUNK)], sems).wait()

            pltpu.sync_copy(eidx_hbm.at[w, j], ib)
            for g in range(G):
                pltpu.async_copy(h_hbm.at[ib.at[g]],
                                 gb.at[pl.ds(g * CHUNK, CHUNK)], semg)
            for g in range(G):
                pltpu.make_async_copy(
                    h_hbm.at[ib.at[g]],
                    gb.at[pl.ds(g * CHUNK, CHUNK)], semg).wait()
            for g in range(G):
                pltpu.async_copy(gb.at[pl.ds(g * CHUNK, CHUNK)],
                                 acc.at[ib.at[G + g]], sems, add=True)

        @pl.loop(0, nblk)
        def _(j):
            @pl.when(j % 2 == 0)
            def _():
                block(j, ib0, gb0, semg0, sems0)

            @pl.when(j % 2 == 1)
            def _():
                block(j, ib1, gb1, semg1, sems1)

        for gb, sems in ((gb0, sems0), (gb1, sems1)):
            for g in range(G):
                pltpu.make_async_copy(
                    h_hbm.at[pl.ds(0, CHUNK)],
                    gb.at[pl.ds(g * CHUNK, CHUNK)], sems).wait()

        plsc.subcore_barrier()

        @pl.loop(0, 8)
        def _(t):
            off = s * NPT + t * (NPT // 8)
            pltpu.sync_copy(acc.at[pl.ds(off, NPT // 8)],
                            gb0.at[pl.ds(0, NPT // 8)])
            pltpu.sync_copy(gb0.at[pl.ds(0, NPT // 8)],
                            out_hbm.at[c, pl.ds(off, NPT // 8)])

    return body


def _make_prop2_body(nblk):
    def body(eidx_hbm, h2f_hbm, out_hbm,
             ib0, ib1, jb0, jb1, gb0, gb1, bb, acc,
             semg0, semg1, sems0, sems1):
        c, s = _worker()
        w = c * NS + s
        npt2 = 2 * NP // NS
        _fill(bb, npt2 // 8, 0.0, jnp.float32)

        @pl.loop(0, 8)
        def _(t):
            pltpu.sync_copy(bb, acc.at[pl.ds(s * npt2 + t * (npt2 // 8),
                                             npt2 // 8)])

        plsc.subcore_barrier()

        def block(j, ib, jb, gb, semg, sems):
            @pl.when(j >= 2)
            def _():
                for g in range(2 * G):
                    pltpu.make_async_copy(
                        h2f_hbm.at[pl.ds(0, CHUNK)],
                        gb.at[pl.ds(g * CHUNK, CHUNK)], sems).wait()

            pltpu.sync_copy(eidx_hbm.at[w, j], ib)
            for g in range(G):
                for k in range(CHUNK // 16):
                    sl = pl.ds(k * 16, 16)
                    rv = ib[g, sl] * 2
                    jb[g, sl] = rv
                    jb[G + g, sl] = rv + 1
                    cv = ib[G + g, sl] * 2
                    jb[2 * G + g, sl] = cv
                    jb[3 * G + g, sl] = cv + 1
            for g in range(2 * G):
                pltpu.async_copy(h2f_hbm.at[jb.at[g]],
                                 gb.at[pl.ds(g * CHUNK, CHUNK)], semg)
            for g in range(2 * G):
                pltpu.make_async_copy(
                    h2f_hbm.at[jb.at[g]],
                    gb.at[pl.ds(g * CHUNK, CHUNK)], semg).wait()
            for g in range(2 * G):
                pltpu.async_copy(gb.at[pl.ds(g * CHUNK, CHUNK)],
                                 acc.at[jb.at[2 * G + g]], sems, add=True)

        @pl.loop(0, nblk)
        def _(j):
            @pl.when(j % 2 == 0)
            def _():
                block(j, ib0, jb0, gb0, semg0, sems0)

            @pl.when(j % 2 == 1)
            def _():
                block(j, ib1, jb1, gb1, semg1, sems1)

        for gb, sems in ((gb0, sems0), (gb1, sems1)):
            for g in range(2 * G):
                pltpu.make_async_copy(
                    h2f_hbm.at[pl.ds(0, CHUNK)],
                    gb.at[pl.ds(g * CHUNK, CHUNK)], sems).wait()

        plsc.subcore_barrier()

        @pl.loop(0, 8)
        def _(t):
            off = s * npt2 + t * (npt2 // 8)
            pltpu.sync_copy(acc.at[pl.ds(off, npt2 // 8)], bb)
            pltpu.sync_copy(bb, out_hbm.at[c, pl.ds(off, npt2 // 8)])

    return body


def _prop2_kernel(nblk):
    f32 = jnp.float32
    return pl.kernel(
        _make_prop2_body(nblk),
        out_type=jax.ShapeDtypeStruct((NC, 2 * NP), f32),
        mesh=_mesh(),
        compiler_params=_SC_PARAMS,
        scratch_types=[
            pltpu.VMEM((2 * G, CHUNK), jnp.int32),
            pltpu.VMEM((2 * G, CHUNK), jnp.int32),
            pltpu.VMEM((4 * G, CHUNK), jnp.int32),
            pltpu.VMEM((4 * G, CHUNK), jnp.int32),
            pltpu.VMEM((2 * G * CHUNK,), f32),
            pltpu.VMEM((2 * G * CHUNK,), f32),
            pltpu.VMEM((2 * NP // NS // 8,), f32),
            pltpu.VMEM_SHARED((2 * NP,), f32),
            pltpu.SemaphoreType.DMA,
            pltpu.SemaphoreType.DMA,
            pltpu.SemaphoreType.DMA,
            pltpu.SemaphoreType.DMA,
        ],
    )


def _prop_kernel(nblk, d):
    f32 = jnp.float32
    return pl.kernel(
        _make_prop_body(nblk, d),
        out_type=jax.ShapeDtypeStruct((NC, NP, d), f32),
        mesh=_mesh(),
        compiler_params=_SC_PARAMS,
        scratch_types=[
            pltpu.VMEM((2 * G, CHUNK), jnp.int32),
            pltpu.VMEM((2 * G, CHUNK), jnp.int32),
            pltpu.VMEM((G * CHUNK, d), f32),
            pltpu.VMEM((G * CHUNK, d), f32),
            pltpu.VMEM_SHARED((NP, d), f32),
            pltpu.SemaphoreType.DMA,
            pltpu.SemaphoreType.DMA,
            pltpu.SemaphoreType.DMA,
            pltpu.SemaphoreType.DMA,
        ],
    )


def _k2_body(x_ref, w1_ref, dinv_ref, o_ref):
    t1 = jnp.dot(x_ref[...], w1_ref[...], preferred_element_type=jnp.float32)
    o_ref[...] = dinv_ref[...] * t1


def _k4_body(acc_ref, h1p_ref, dinv_ref, b1_ref, w2_ref, o_ref):
    dinv = dinv_ref[...]
    agg = acc_ref[0] + acc_ref[1] + h1p_ref[...]
    z = jnp.maximum(dinv * agg + b1_ref[...], 0.0)
    t2 = jnp.dot(z, w2_ref[...], preferred_element_type=jnp.float32)
    o_ref[...] = dinv[:, :CLS] * t2


def _k6_body(acc_ref, h2p_ref, dinv_ref, b2_ref, o_ref):
    o = dinv_ref[...][:, :CLS] * (acc_ref[0] + acc_ref[1] + h2p_ref[...])
    o = o + b2_ref[...]
    m = jnp.max(o, axis=1, keepdims=True)
    ssum = jnp.sum(jnp.exp(o - m), axis=1, keepdims=True)
    o_ref[...] = o - m - jnp.log(ssum)


@jax.jit
def kernel(x, edge_index, W1, b1, W2, b2):
    f32 = jnp.float32
    row = edge_index[0].astype(jnp.int32)
    col = edge_index[1].astype(jnp.int32)
    e = row.shape[0]
    blk_edges = NW * G * CHUNK
    nblk = (e + blk_edges - 1) // blk_edges
    ep = nblk * blk_edges
    npad = ep - e
    padi = N + (jnp.arange(npad, dtype=jnp.int32) % 128)
    row_p = jnp.concatenate([row, padi]).reshape(NW, nblk, G, CHUNK)
    col_p = jnp.concatenate([col, padi]).reshape(NW, nblk, G, CHUNK)
    eidx = jnp.stack([row_p, col_p], axis=2).reshape(NW, nblk, 2 * G, CHUNK)
    x_p = jnp.pad(x, ((0, NP - N), (0, 0)))
    z1 = jnp.zeros((NP,), f32)
    z16 = jnp.zeros((NP, H), f32)

    k1a = pl.kernel(
        functools.partial(_k1a_body, nblk),
        out_type=jax.ShapeDtypeStruct((NC, NP), f32),
        mesh=_mesh(),
        compiler_params=_SC_PARAMS,
        scratch_types=[
            pltpu.VMEM((2 * G, CHUNK), jnp.int32),
            pltpu.VMEM((2 * G, CHUNK), jnp.int32),
            pltpu.VMEM((CHUNK,), f32),
            pltpu.VMEM((NPT // 8,), f32),
            pltpu.VMEM_SHARED((NP,), f32),
            pltpu.SemaphoreType.DMA,
            pltpu.SemaphoreType.DMA,
        ],
    )
    deg_part = k1a(eidx, z1)

    k1b = pl.kernel(
        _k1b_body,
        out_type=jax.ShapeDtypeStruct((NP, H), f32),
        mesh=_mesh(),
        compiler_params=_SC_PARAMS,
        scratch_types=[
            pltpu.VMEM((NPW,), f32),
            pltpu.VMEM((NPW,), f32),
            pltpu.VMEM((16,), f32),
            pltpu.VMEM((NPW, H), f32),
        ],
    )
    dinv16 = k1b(deg_part)

    h1p = pl.pallas_call(
        _k2_body, grid=(NP // BN,),
        in_specs=[pl.BlockSpec((BN, F), lambda i: (i, 0)),
                  pl.BlockSpec((F, H), lambda i: (0, 0)),
                  pl.BlockSpec((BN, H), lambda i: (i, 0))],
        out_specs=pl.BlockSpec((BN, H), lambda i: (i, 0)),
        out_shape=jax.ShapeDtypeStruct((NP, H), f32),
    )(x_p, W1, dinv16)

    acc1 = _prop_kernel(nblk, H)(eidx, z16, h1p)

    h2p = pl.pallas_call(
        _k4_body, grid=(NP // BN,),
        in_specs=[pl.BlockSpec((NC, BN, H), lambda i: (0, i, 0)),
                  pl.BlockSpec((BN, H), lambda i: (i, 0)),
                  pl.BlockSpec((BN, H), lambda i: (i, 0)),
                  pl.BlockSpec((1, H), lambda i: (0, 0)),
                  pl.BlockSpec((H, CLS), lambda i: (0, 0))],
        out_specs=pl.BlockSpec((BN, CLS), lambda i: (i, 0)),
        out_shape=jax.ShapeDtypeStruct((NP, CLS), f32),
    )(acc1, h1p, dinv16, b1.reshape(1, H), W2)

    acc2 = _prop2_kernel(nblk)(eidx, h2p.reshape(-1)).reshape(NC, NP, CLS)

    out = pl.pallas_call(
        _k6_body, grid=(NP // BN,),
        in_specs=[pl.BlockSpec((NC, BN, CLS), lambda i: (0, i, 0)),
                  pl.BlockSpec((BN, CLS), lambda i: (i, 0)),
                  pl.BlockSpec((BN, H), lambda i: (i, 0)),
                  pl.BlockSpec((1, CLS), lambda i: (0, 0))],
        out_specs=pl.BlockSpec((BN, CLS), lambda i: (i, 0)),
        out_shape=jax.ShapeDtypeStruct((NP, CLS), f32),
    )(acc2, h2p, dinv16, b2.reshape(1, CLS))

    return out[:N]
